# baseline (speedup 1.0000x reference)
import functools

import jax
import jax.numpy as jnp
from jax import lax
from jax.experimental import pallas as pl
from jax.experimental.pallas import tpu as pltpu
from jax.experimental.pallas import tpu_sc as plsc

V = 1000000
D = 64
B = 4096 * 50
NC, NS = 2, 16
NW = NC * NS
BPW = B // NW
W = 8192
SHIFT = 13
VCUT = (V // W) * W
VTAIL = VCUT + 512
NSLAB = 124
NSLAB_PAD = 144
HB = 64
NVREG = BPW // 16


@functools.lru_cache(maxsize=None)
def _build():
    mesh = plsc.VectorSubcoreMesh(core_axis_name="c", subcore_axis_name="s")

    @functools.partial(
        pl.kernel,
        out_type=(
            jax.ShapeDtypeStruct((NW * (BPW + HB) * D,), jnp.float32),
            jax.ShapeDtypeStruct((B,), jnp.int32),
        ),
        mesh=mesh,
        compiler_params=pltpu.CompilerParams(
            use_tc_tiling_on_sc=True, needs_layout_passes=False),
        scratch_types=[
            pltpu.VMEM((BPW,), jnp.int32),
            pltpu.VMEM((BPW + HB,), jnp.int32),
            pltpu.VMEM((BPW,), jnp.int32),
            pltpu.VMEM((16 * 128,), jnp.int32),
            pltpu.VMEM((16 * 128,), jnp.int32),
            pltpu.VMEM((NSLAB_PAD,), jnp.int32),
            pltpu.VMEM((HB * D,), jnp.int32),
            pltpu.VMEM((HB * D,), jnp.float32),
            pltpu.VMEM_SHARED((D * W,), jnp.float32),
            pltpu.VMEM_SHARED((D * W,), jnp.float32),
            pltpu.SemaphoreType.DMA,
            pltpu.SemaphoreType.DMA,
            pltpu.SemaphoreType.DMA,
        ],
    )
    def body(wt, wt_tail, idx, stg, pjo, idx_v, srt_c, posj_v, cnt16, cur16,
             bnd, eidx, rows, slab0, slab1, semf, semg, semo):
        cid = lax.axis_index("c")
        sid = lax.axis_index("s")
        wid = sid * NC + cid
        base = wid * BPW
        sbase = wid * (BPW + HB)
        iota = lax.broadcasted_iota(jnp.int32, (16,), 0)
        ones = jnp.ones((16,), jnp.int32)
        zeros = jnp.zeros((16,), jnp.int32)

        pltpu.sync_copy(idx.at[pl.ds(base, BPW)], idx_v)

        for i in range(128):
            cnt16[pl.ds(i * 16, 16)] = zeros

        def slab_of(v):
            return (v >> SHIFT) + jnp.where(v >= VTAIL, 1, 0)

        def pass_a(i, _):
            v = idx_v[pl.ds(i * 16, 16)]
            s = slab_of(v)
            plsc.addupdate_scatter(cnt16, [iota * 128 + s], ones)
            return 0

        lax.fori_loop(0, NVREG, pass_a, 0)

        carry = jnp.int32(0)
        for ch in range(8):
            tot = cnt16[pl.ds(ch * 16, 16)]
            for l in range(1, 16):
                tot = tot + cnt16[pl.ds(l * 128 + ch * 16, 16)]
            inc = plsc.cumsum(tot)
            excl = inc - tot + carry
            bnd[pl.ds(ch * 16, 16)] = excl
            carry = carry + inc[15]
        bnd[pl.ds(128, 16)] = jnp.broadcast_to(carry, (16,))

        for ch in range(8):
            acc = bnd[pl.ds(ch * 16, 16)]
            for l in range(16):
                cur16[pl.ds(l * 128 + ch * 16, 16)] = acc
                acc = acc + cnt16[pl.ds(l * 128 + ch * 16, 16)]

        def pass_b(i, _):
            v = idx_v[pl.ds(i * 16, 16)]
            s = slab_of(v)
            c = v & (W - 1)
            c = jnp.where(s == NSLAB - 1, c - 512, c)
            eix = iota * 128 + s
            cur = plsc.load_gather(cur16, [eix])
            plsc.store_scatter(srt_c, [cur], c)
            posj_v[pl.ds(i * 16, 16)] = sbase + cur
            plsc.store_scatter(cur16, [eix], cur + 1)
            return 0

        lax.fori_loop(0, NVREG, pass_b, 0)
        pltpu.sync_copy(posj_v, pjo.at[pl.ds(base, BPW)])

        def fill_descs(g, buf, width, src, col0):
            descs = []
            for dd in range(4):
                d = sid * 4 + dd
                descs.append(pltpu.make_async_copy(
                    src.at[d, pl.ds(col0, width)],
                    buf.at[pl.ds(d * W, width)], semf))
            return descs

        def all_fill_descs(g, buf):
            full = fill_descs(g, buf, W, wt, g * W)
            mid = fill_descs(g, buf, 512, wt, VCUT)
            tail = fill_descs(g, buf, 128, wt_tail, 0)
            return full, mid, tail

        def issue_fill(g, buf):
            full, mid, tail = all_fill_descs(g, buf)

            @pl.when(g < NSLAB - 2)
            def _():
                for dsc in full:
                    dsc.start()

            @pl.when(g == NSLAB - 2)
            def _():
                for dsc in mid:
                    dsc.start()

            @pl.when(g == NSLAB - 1)
            def _():
                for dsc in tail:
                    dsc.start()

        def wait_fill(g, buf):
            full, mid, tail = all_fill_descs(g, buf)

            @pl.when(g < NSLAB - 2)
            def _():
                for dsc in full:
                    dsc.wait()

            @pl.when(g == NSLAB - 2)
            def _():
                for dsc in mid:
                    dsc.wait()

            @pl.when(g == NSLAB - 1)
            def _():
                for dsc in tail:
                    dsc.wait()

        issue_fill(0, slab0)

        dvecs = [(iota + dd * 16) * W for dd in range(4)]

        def stg_desc(p):
            return pltpu.make_async_copy(
                rows, stg.at[pl.ds((sbase + p) * D, HB * D)], semo)

        def consume(g, buf):
            bb = bnd[pl.ds(g, 16)]
            lo = bb[0]
            hi = bb[1]
            nb = (hi - lo + (HB - 1)) // HB

            def batch(bi, _):
                p = lo + bi * HB

                @pl.when(bi > 0)
                def _():
                    stg_desc(p - HB).wait()

                for sub in range(4):
                    cv = srt_c[pl.ds(p + sub * 16, 16)]
                    pos = p + sub * 16 + iota
                    cq = jnp.where(pos < hi, cv, 0)
                    for h in range(16):
                        cb = jnp.broadcast_to(cq[h], (16,))
                        hh = sub * 16 + h
                        for dd in range(4):
                            eidx[pl.ds(hh * D + dd * 16, 16)] = cb + dvecs[dd]
                pltpu.async_copy(buf.at[eidx], rows, semg).wait()
                stg_desc(p).start()
                return 0

            lax.fori_loop(0, nb, batch, 0)

            @pl.when(nb > 0)
            def _():
                stg_desc(lo + (nb - 1) * HB).wait()

        def slab_step(g, _):
            @pl.when(g % 2 == 0)
            def _():
                wait_fill(g, slab0)

            @pl.when(g % 2 == 1)
            def _():
                wait_fill(g, slab1)

            plsc.subcore_barrier()

            @pl.when(jnp.logical_and(g + 1 < NSLAB, g % 2 == 0))
            def _():
                issue_fill(g + 1, slab1)

            @pl.when(jnp.logical_and(g + 1 < NSLAB, g % 2 == 1))
            def _():
                issue_fill(g + 1, slab0)

            @pl.when(g % 2 == 0)
            def _():
                consume(g, slab0)

            @pl.when(g % 2 == 1)
            def _():
                consume(g, slab1)

            return 0

        lax.fori_loop(0, NSLAB, slab_step, 0)

    return body


CHUNK = 800
NCHUNK = BPW // CHUNK


@functools.lru_cache(maxsize=None)
def _build_permute():
    mesh = plsc.VectorSubcoreMesh(core_axis_name="c", subcore_axis_name="s")

    @functools.partial(
        pl.kernel,
        out_type=jax.ShapeDtypeStruct((B, D), jnp.float32),
        mesh=mesh,
        compiler_params=pltpu.CompilerParams(use_tc_tiling_on_sc=False),
        scratch_types=[
            pltpu.VMEM((BPW,), jnp.int32),
            pltpu.VMEM((CHUNK, D), jnp.float32),
            pltpu.VMEM((CHUNK, D), jnp.float32),
            pltpu.SemaphoreType.DMA,
            pltpu.SemaphoreType.DMA,
        ],
    )
    def permute(stg2d, pjo, out, pos_v, rows0, rows1, sem0, sem1):
        wid = lax.axis_index("s") * NC + lax.axis_index("c")
        base = wid * BPW
        pltpu.sync_copy(pjo.at[pl.ds(base, BPW)], pos_v)

        rows = (rows0, rows1)
        sems = (sem0, sem1)
        pending = [None, None]
        pending[0] = pltpu.async_copy(
            stg2d.at[pos_v.at[pl.ds(0, CHUNK)]], rows0, sem0)
        for g in range(NCHUNK):
            bsl = g % 2
            pending[bsl].wait()
            if g + 1 < NCHUNK:
                nb2 = 1 - bsl
                pending[nb2] = pltpu.async_copy(
                    stg2d.at[pos_v.at[pl.ds((g + 1) * CHUNK, CHUNK)]],
                    rows[nb2], sems[nb2])
            pltpu.sync_copy(
                rows[bsl], out.at[pl.ds(base + g * CHUNK, CHUNK)])

    return permute


def kernel(src_seq, emb_weight):
    idx = src_seq.reshape(-1).astype(jnp.int32)
    wt = emb_weight.T
    wt_tail = jnp.pad(emb_weight[VTAIL:], ((0, 64), (0, 0))).T
    stg, pjo = _build()(wt, wt_tail, idx)
    flat = _build_permute()(stg.reshape(NW * (BPW + HB), D), pjo)
    return flat.reshape(src_seq.shape + (D,))


# speedup vs baseline: 2.0784x; 2.0784x over previous
import functools

import jax
import jax.numpy as jnp
from jax import lax
from jax.experimental import pallas as pl
from jax.experimental.pallas import tpu as pltpu
from jax.experimental.pallas import tpu_sc as plsc

V = 1000000
D = 64
DP = 128
B = 4096 * 50
NC, NS = 2, 16
NW = NC * NS
BPW = B // NW
CHUNK = 400
NCHUNK = BPW // CHUNK


@functools.lru_cache(maxsize=None)
def _build():
    mesh = plsc.VectorSubcoreMesh(core_axis_name="c", subcore_axis_name="s")

    @functools.partial(
        pl.kernel,
        out_type=jax.ShapeDtypeStruct((B, D), jnp.float32),
        mesh=mesh,
        compiler_params=pltpu.CompilerParams(use_tc_tiling_on_sc=False),
        scratch_types=[
            pltpu.VMEM((BPW,), jnp.int32),
            pltpu.VMEM((CHUNK, DP), jnp.float32),
            pltpu.VMEM((CHUNK, DP), jnp.float32),
            pltpu.SemaphoreType.DMA,
            pltpu.SemaphoreType.DMA,
        ],
    )
    def gather_kernel(table_hbm, idx_hbm, out_hbm, idx_v, rows0, rows1,
                      sem0, sem1):
        wid = lax.axis_index("s") * NC + lax.axis_index("c")
        base = wid * BPW
        pltpu.sync_copy(idx_hbm.at[pl.ds(base, BPW)], idx_v)

        rows = (rows0, rows1)
        sems = (sem0, sem1)
        pending = [None, None]
        pending[0] = pltpu.async_copy(
            table_hbm.at[idx_v.at[pl.ds(0, CHUNK)]], rows0, sem0)
        for g in range(NCHUNK):
            b = g % 2
            pending[b].wait()
            if g + 1 < NCHUNK:
                nb = 1 - b
                pending[nb] = pltpu.async_copy(
                    table_hbm.at[idx_v.at[pl.ds((g + 1) * CHUNK, CHUNK)]],
                    rows[nb], sems[nb])
            pltpu.sync_copy(
                rows[b].at[:, pl.ds(0, D)],
                out_hbm.at[pl.ds(base + g * CHUNK, CHUNK)])

    return gather_kernel


def kernel(src_seq, emb_weight):
    idx = src_seq.reshape(-1).astype(jnp.int32)
    wt128 = jnp.pad(emb_weight, ((0, 0), (0, DP - D)))
    out = _build()(wt128, idx)
    return out.reshape(src_seq.shape + (emb_weight.shape[-1],))


# speedup vs baseline: 2.2784x; 1.0962x over previous
import functools

import jax
import jax.numpy as jnp
from jax import lax
from jax.experimental import pallas as pl
from jax.experimental.pallas import tpu as pltpu
from jax.experimental.pallas import tpu_sc as plsc

V = 1000000
D = 64
DP = 128
B = 4096 * 50
NC, NS = 2, 16
NW = NC * NS
BPW = B // NW
CHUNK = 400
NCHUNK = BPW // CHUNK


@functools.lru_cache(maxsize=None)
def _build():
    mesh = plsc.VectorSubcoreMesh(core_axis_name="c", subcore_axis_name="s")

    @functools.partial(
        pl.kernel,
        out_type=jax.ShapeDtypeStruct((B, D), jnp.float32),
        mesh=mesh,
        compiler_params=pltpu.CompilerParams(use_tc_tiling_on_sc=False),
        scratch_types=[
            pltpu.VMEM((BPW,), jnp.int32),
            pltpu.VMEM((CHUNK, DP), jnp.float32),
            pltpu.VMEM((CHUNK, DP), jnp.float32),
            pltpu.SemaphoreType.DMA,
            pltpu.SemaphoreType.DMA,
        ],
    )
    def gather_kernel(table_hbm, idx_hbm, out_hbm, idx_v, rows0, rows1,
                      sem0, sem1):
        wid = lax.axis_index("s") * NC + lax.axis_index("c")
        base = wid * BPW
        pltpu.sync_copy(idx_hbm.at[pl.ds(base, BPW)], idx_v)

        rows = (rows0, rows1)
        sems = (sem0, sem1)
        pending = [None, None]
        pending[0] = pltpu.async_copy(
            table_hbm.at[idx_v.at[pl.ds(0, CHUNK)]], rows0, sem0)
        for g in range(NCHUNK):
            b = g % 2
            pending[b].wait()
            if g + 1 < NCHUNK:
                nb = 1 - b
                pending[nb] = pltpu.async_copy(
                    table_hbm.at[idx_v.at[pl.ds((g + 1) * CHUNK, CHUNK)]],
                    rows[nb], sems[nb])
            pltpu.sync_copy(
                rows[b].at[:, pl.ds(0, D)],
                out_hbm.at[pl.ds(base + g * CHUNK, CHUNK)])

    return gather_kernel


BLK = 2048


@functools.lru_cache(maxsize=None)
def _build_transpose():
    def tbody(x_ref, o_ref):
        xt = x_ref[...].T
        o_ref[...] = jnp.concatenate(
            [xt, jnp.zeros((BLK, DP - D), jnp.float32)], axis=1)

    return pl.pallas_call(
        tbody,
        grid=(pl.cdiv(V, BLK),),
        in_specs=[pl.BlockSpec((D, BLK), lambda i: (0, i))],
        out_specs=pl.BlockSpec((BLK, DP), lambda i: (i, 0)),
        out_shape=jax.ShapeDtypeStruct((V, DP), jnp.float32),
    )


def kernel(src_seq, emb_weight):
    idx = src_seq.reshape(-1).astype(jnp.int32)
    wt128 = _build_transpose()(emb_weight.T)
    out = _build()(wt128, idx)
    return out.reshape(src_seq.shape + (emb_weight.shape[-1],))


# speedup vs baseline: 2.2791x; 1.0003x over previous
import functools

import jax
import jax.numpy as jnp
from jax import lax
from jax.experimental import pallas as pl
from jax.experimental.pallas import tpu as pltpu
from jax.experimental.pallas import tpu_sc as plsc

V = 1000000
D = 64
DP = 128
B = 4096 * 50
NC, NS = 2, 16
NW = NC * NS
BPW = B // NW
CHUNK = 400
NCHUNK = BPW // CHUNK


@functools.lru_cache(maxsize=None)
def _build():
    mesh = plsc.VectorSubcoreMesh(core_axis_name="c", subcore_axis_name="s")

    @functools.partial(
        pl.kernel,
        out_type=jax.ShapeDtypeStruct((B, D), jnp.float32),
        mesh=mesh,
        compiler_params=pltpu.CompilerParams(use_tc_tiling_on_sc=False),
        scratch_types=[
            pltpu.VMEM((BPW,), jnp.int32),
            pltpu.VMEM((CHUNK, DP), jnp.float32),
            pltpu.VMEM((CHUNK, DP), jnp.float32),
            pltpu.SemaphoreType.DMA,
            pltpu.SemaphoreType.DMA,
        ],
    )
    def gather_kernel(table_hbm, idx_hbm, out_hbm, idx_v, rows0, rows1,
                      sem0, sem1):
        wid = lax.axis_index("s") * NC + lax.axis_index("c")
        base = wid * BPW
        pltpu.sync_copy(idx_hbm.at[pl.ds(base, BPW)], idx_v)

        rows = (rows0, rows1)
        sems = (sem0, sem1)
        pending = [None, None]
        pending[0] = pltpu.async_copy(
            table_hbm.at[idx_v.at[pl.ds(0, CHUNK)]], rows0, sem0)
        for g in range(NCHUNK):
            b = g % 2
            pending[b].wait()
            if g + 1 < NCHUNK:
                nb = 1 - b
                pending[nb] = pltpu.async_copy(
                    table_hbm.at[idx_v.at[pl.ds((g + 1) * CHUNK, CHUNK)]],
                    rows[nb], sems[nb])
            pltpu.sync_copy(
                rows[b].at[:, pl.ds(0, D)],
                out_hbm.at[pl.ds(base + g * CHUNK, CHUNK)])

    return gather_kernel


BLK = 2048


@functools.lru_cache(maxsize=None)
def _build_transpose():
    def tbody(x_ref, o_ref):
        o_ref[:, pl.ds(0, D)] = x_ref[...].T

    return pl.pallas_call(
        tbody,
        grid=(pl.cdiv(V, BLK),),
        in_specs=[pl.BlockSpec((D, BLK), lambda i: (0, i))],
        out_specs=pl.BlockSpec((BLK, DP), lambda i: (i, 0)),
        out_shape=jax.ShapeDtypeStruct((V, DP), jnp.float32),
    )


def kernel(src_seq, emb_weight):
    idx = src_seq.reshape(-1).astype(jnp.int32)
    wt128 = _build_transpose()(emb_weight.T)
    out = _build()(wt128, idx)
    return out.reshape(src_seq.shape + (emb_weight.shape[-1],))


# speedup vs baseline: 3.2172x; 1.4116x over previous
import functools

import jax
import jax.numpy as jnp
from jax import lax
from jax.experimental import pallas as pl
from jax.experimental.pallas import tpu as pltpu
from jax.experimental.pallas import tpu_sc as plsc

V = 1000000
D = 64
DP = 128
B = 4096 * 50
NC, NS = 2, 16
NW = NC * NS
BPW = B // NW
CHUNK = 400
NCHUNK = BPW // CHUNK


@functools.lru_cache(maxsize=None)
def _build():
    mesh = plsc.VectorSubcoreMesh(core_axis_name="c", subcore_axis_name="s")

    @functools.partial(
        pl.kernel,
        out_type=jax.ShapeDtypeStruct((B, D), jnp.float32),
        mesh=mesh,
        compiler_params=pltpu.CompilerParams(use_tc_tiling_on_sc=False),
        scratch_types=[
            pltpu.VMEM((BPW,), jnp.int32),
            pltpu.VMEM((CHUNK, DP), jnp.float32),
            pltpu.VMEM((CHUNK, DP), jnp.float32),
            pltpu.SemaphoreType.DMA,
            pltpu.SemaphoreType.DMA,
        ],
    )
    def gather_kernel(table_hbm, idx_hbm, out_hbm, idx_v, rows0, rows1,
                      sem0, sem1):
        wid = lax.axis_index("s") * NC + lax.axis_index("c")
        base = wid * BPW
        pltpu.sync_copy(idx_hbm.at[pl.ds(base, BPW)], idx_v)

        rows = (rows0, rows1)
        sems = (sem0, sem1)
        pending = [None, None]
        pending[0] = pltpu.async_copy(
            table_hbm.at[idx_v.at[pl.ds(0, CHUNK)]], rows0, sem0)
        for g in range(NCHUNK):
            b = g % 2
            pending[b].wait()
            if g + 1 < NCHUNK:
                nb = 1 - b
                pending[nb] = pltpu.async_copy(
                    table_hbm.at[idx_v.at[pl.ds((g + 1) * CHUNK, CHUNK)]],
                    rows[nb], sems[nb])
            pltpu.sync_copy(
                rows[b].at[:, pl.ds(0, D)],
                out_hbm.at[pl.ds(base + g * CHUNK, CHUNK)])

    return gather_kernel


BLK = 8192


@functools.lru_cache(maxsize=None)
def _build_transpose():
    def tbody(x_ref, o_ref):
        o_ref[:, pl.ds(0, D)] = x_ref[...].T

    return pl.pallas_call(
        tbody,
        grid=(pl.cdiv(V, BLK),),
        in_specs=[pl.BlockSpec((D, BLK), lambda i: (0, i))],
        out_specs=pl.BlockSpec((BLK, DP), lambda i: (i, 0)),
        out_shape=jax.ShapeDtypeStruct((V, DP), jnp.float32),
    )


def kernel(src_seq, emb_weight):
    idx = src_seq.reshape(-1).astype(jnp.int32)
    wt128 = _build_transpose()(emb_weight.T)
    out = _build()(wt128, idx)
    return out.reshape(src_seq.shape + (emb_weight.shape[-1],))


# speedup vs baseline: 3.3483x; 1.0408x over previous
import functools

import jax
import jax.numpy as jnp
from jax import lax
from jax.experimental import pallas as pl
from jax.experimental.pallas import tpu as pltpu
from jax.experimental.pallas import tpu_sc as plsc

V = 1000000
D = 64
DP = 128
B = 4096 * 50
NC, NS = 2, 16
NW = NC * NS
BPW = B // NW
CHUNK = 400
NCHUNK = BPW // CHUNK


@functools.lru_cache(maxsize=None)
def _build():
    mesh = plsc.VectorSubcoreMesh(core_axis_name="c", subcore_axis_name="s")

    @functools.partial(
        pl.kernel,
        out_type=jax.ShapeDtypeStruct((B, D), jnp.float32),
        mesh=mesh,
        compiler_params=pltpu.CompilerParams(use_tc_tiling_on_sc=False),
        scratch_types=[
            pltpu.VMEM((BPW,), jnp.int32),
            pltpu.VMEM((CHUNK, DP), jnp.float32),
            pltpu.VMEM((CHUNK, DP), jnp.float32),
            pltpu.SemaphoreType.DMA,
            pltpu.SemaphoreType.DMA,
        ],
    )
    def gather_kernel(table_hbm, idx_hbm, out_hbm, idx_v, rows0, rows1,
                      sem0, sem1):
        wid = lax.axis_index("s") * NC + lax.axis_index("c")
        base = wid * BPW
        pltpu.sync_copy(idx_hbm.at[pl.ds(base, BPW)], idx_v)

        rows = (rows0, rows1)
        sems = (sem0, sem1)
        pending = [None, None]
        pending[0] = pltpu.async_copy(
            table_hbm.at[idx_v.at[pl.ds(0, CHUNK)]], rows0, sem0)
        for g in range(NCHUNK):
            b = g % 2
            pending[b].wait()
            if g + 1 < NCHUNK:
                nb = 1 - b
                pending[nb] = pltpu.async_copy(
                    table_hbm.at[idx_v.at[pl.ds((g + 1) * CHUNK, CHUNK)]],
                    rows[nb], sems[nb])
            pltpu.sync_copy(
                rows[b].at[:, pl.ds(0, D)],
                out_hbm.at[pl.ds(base + g * CHUNK, CHUNK)])

    return gather_kernel


BLK = 16384


@functools.lru_cache(maxsize=None)
def _build_transpose():
    def tbody(x_ref, o_ref):
        o_ref[:, pl.ds(0, D)] = x_ref[...].T

    return pl.pallas_call(
        tbody,
        grid=(pl.cdiv(V, BLK),),
        in_specs=[pl.BlockSpec((D, BLK), lambda i: (0, i))],
        out_specs=pl.BlockSpec((BLK, DP), lambda i: (i, 0)),
        out_shape=jax.ShapeDtypeStruct((V, DP), jnp.float32),
    )


def kernel(src_seq, emb_weight):
    idx = src_seq.reshape(-1).astype(jnp.int32)
    wt128 = _build_transpose()(emb_weight.T)
    out = _build()(wt128, idx)
    return out.reshape(src_seq.shape + (emb_weight.shape[-1],))


# speedup vs baseline: 3.5448x; 1.0587x over previous
import functools

import jax
import jax.numpy as jnp
from jax import lax
from jax.experimental import pallas as pl
from jax.experimental.pallas import tpu as pltpu
from jax.experimental.pallas import tpu_sc as plsc

V = 1000000
D = 64
DP = 128
B = 4096 * 50
NC, NS = 2, 16
NW = NC * NS
BPW = B // NW
CHUNK = 400
NCHUNK = BPW // CHUNK


@functools.lru_cache(maxsize=None)
def _build():
    mesh = plsc.VectorSubcoreMesh(core_axis_name="c", subcore_axis_name="s")

    @functools.partial(
        pl.kernel,
        out_type=jax.ShapeDtypeStruct((B, D), jnp.float32),
        mesh=mesh,
        compiler_params=pltpu.CompilerParams(use_tc_tiling_on_sc=False),
        scratch_types=[
            pltpu.VMEM((BPW,), jnp.int32),
            pltpu.VMEM((CHUNK, D), jnp.float32),
            pltpu.VMEM((CHUNK, D), jnp.float32),
            pltpu.SemaphoreType.DMA,
            pltpu.SemaphoreType.DMA,
        ],
    )
    def gather_kernel(table_hbm, idx_hbm, out_hbm, idx_v, rows0, rows1,
                      sem0, sem1):
        wid = lax.axis_index("s") * NC + lax.axis_index("c")
        base = wid * BPW
        pltpu.sync_copy(idx_hbm.at[pl.ds(base, BPW)], idx_v)

        rows = (rows0, rows1)
        sems = (sem0, sem1)
        pending = [None, None]
        pending[0] = pltpu.async_copy(
            table_hbm.at[idx_v.at[pl.ds(0, CHUNK)]], rows0, sem0)
        for g in range(NCHUNK):
            b = g % 2
            pending[b].wait()
            if g + 1 < NCHUNK:
                nb = 1 - b
                pending[nb] = pltpu.async_copy(
                    table_hbm.at[idx_v.at[pl.ds((g + 1) * CHUNK, CHUNK)]],
                    rows[nb], sems[nb])
            pltpu.sync_copy(
                rows[b], out_hbm.at[pl.ds(base + g * CHUNK, CHUNK)])

    return gather_kernel


BLK = 16384


@functools.lru_cache(maxsize=None)
def _build_transpose():
    def tbody(x_ref, o_ref):
        o_ref[:, pl.ds(0, D)] = x_ref[...].T

    return pl.pallas_call(
        tbody,
        grid=(pl.cdiv(V, BLK),),
        in_specs=[pl.BlockSpec((D, BLK), lambda i: (0, i))],
        out_specs=pl.BlockSpec((BLK, DP), lambda i: (i, 0)),
        out_shape=jax.ShapeDtypeStruct((V, DP), jnp.float32),
    )


def kernel(src_seq, emb_weight):
    idx2 = src_seq.reshape(-1).astype(jnp.int32) * 2
    wt2v = _build_transpose()(emb_weight.T).reshape(2 * V, D)
    out = _build()(wt2v, idx2)
    return out.reshape(src_seq.shape + (emb_weight.shape[-1],))


# speedup vs baseline: 3.6081x; 1.0179x over previous
import functools

import jax
import jax.numpy as jnp
from jax import lax
from jax.experimental import pallas as pl
from jax.experimental.pallas import tpu as pltpu
from jax.experimental.pallas import tpu_sc as plsc

V = 1000000
D = 64
DP = 128
B = 4096 * 50
NC, NS = 2, 16
NW = NC * NS
BPW = B // NW
CHUNK = 800
NCHUNK = BPW // CHUNK


@functools.lru_cache(maxsize=None)
def _build():
    mesh = plsc.VectorSubcoreMesh(core_axis_name="c", subcore_axis_name="s")

    @functools.partial(
        pl.kernel,
        out_type=jax.ShapeDtypeStruct((B, D), jnp.float32),
        mesh=mesh,
        compiler_params=pltpu.CompilerParams(use_tc_tiling_on_sc=False),
        scratch_types=[
            pltpu.VMEM((BPW,), jnp.int32),
            pltpu.VMEM((CHUNK, D), jnp.float32),
            pltpu.VMEM((CHUNK, D), jnp.float32),
            pltpu.SemaphoreType.DMA,
            pltpu.SemaphoreType.DMA,
        ],
    )
    def gather_kernel(table_hbm, idx_hbm, out_hbm, idx_v, rows0, rows1,
                      sem0, sem1):
        wid = lax.axis_index("s") * NC + lax.axis_index("c")
        base = wid * BPW
        pltpu.sync_copy(idx_hbm.at[pl.ds(base, BPW)], idx_v)

        rows = (rows0, rows1)
        sems = (sem0, sem1)
        pending = [None, None]
        pending[0] = pltpu.async_copy(
            table_hbm.at[idx_v.at[pl.ds(0, CHUNK)]], rows0, sem0)
        for g in range(NCHUNK):
            b = g % 2
            pending[b].wait()
            if g + 1 < NCHUNK:
                nb = 1 - b
                pending[nb] = pltpu.async_copy(
                    table_hbm.at[idx_v.at[pl.ds((g + 1) * CHUNK, CHUNK)]],
                    rows[nb], sems[nb])
            pltpu.sync_copy(
                rows[b], out_hbm.at[pl.ds(base + g * CHUNK, CHUNK)])

    return gather_kernel


BLK = 32768


@functools.lru_cache(maxsize=None)
def _build_transpose():
    def tbody(x_ref, o_ref):
        o_ref[:, pl.ds(0, D)] = x_ref[...].T

    return pl.pallas_call(
        tbody,
        grid=(pl.cdiv(V, BLK),),
        in_specs=[pl.BlockSpec((D, BLK), lambda i: (0, i))],
        out_specs=pl.BlockSpec((BLK, DP), lambda i: (i, 0)),
        out_shape=jax.ShapeDtypeStruct((V, DP), jnp.float32),
    )


def kernel(src_seq, emb_weight):
    idx2 = src_seq.reshape(-1).astype(jnp.int32) * 2
    wt2v = _build_transpose()(emb_weight.T).reshape(2 * V, D)
    out = _build()(wt2v, idx2)
    return out.reshape(src_seq.shape + (emb_weight.shape[-1],))
